# Initial kernel scaffold; baseline (speedup 1.0000x reference)
import jax
import jax.numpy as jnp
from jax.experimental import pallas as pl


def kernel(x, edge_index, batch, W1, b1, W2, b2, Wout, bout):
    raise NotImplementedError("write your pallas kernel here")


# speedup vs baseline: 4.4861x; 4.4861x over previous
import functools

import jax
import jax.numpy as jnp
from jax import lax
from jax.experimental import pallas as pl
from jax.experimental.pallas import tpu as pltpu
from jax.experimental.pallas import tpu_sc as plsc

_N = 10000
_E = 320000
_D = 128
_NG = 16

_NC = 2
_NS = 16
_CHUNK = 128
_CPT = 80
_EPAD = _NC * _NS * _CPT * _CHUNK
_NPAD = _EPAD - _E
_NR = 10240
_RPT = _NR // _NS
_ROWB = _RPT // _CHUNK

_mesh = plsc.VectorSubcoreMesh(
    core_axis_name="c", subcore_axis_name="s", num_cores=_NC, num_subcores=_NS)


@functools.partial(
    pl.kernel,
    out_type=jax.ShapeDtypeStruct((_NC, _NR, _D), jnp.float32),
    mesh=_mesh,
    scratch_types=[
        pltpu.VMEM((_CPT, _CHUNK), jnp.int32),
        pltpu.VMEM((_CHUNK, _D), jnp.float32),
        pltpu.VMEM((_CHUNK, _D), jnp.float32),
        pltpu.VMEM_SHARED((_NR, _D), jnp.float32),
        pltpu.SemaphoreType.DMA,
    ],
)
def _sc_degree(srcb, zeros16, ones16, out, src_v, zbuf, obuf, acc, sem):
    c = lax.axis_index("c")
    s = lax.axis_index("s")
    pltpu.sync_copy(srcb.at[c, s], src_v)
    pltpu.sync_copy(zeros16, zbuf)
    pltpu.sync_copy(ones16, obuf)
    for t in range(_ROWB):
        pltpu.sync_copy(zbuf, acc.at[pl.ds(s * _RPT + t * _CHUNK, _CHUNK)])
    plsc.subcore_barrier()

    def body(j, carry):
        pltpu.sync_copy(obuf, acc.at[src_v.at[j]], add=True)
        return carry

    lax.fori_loop(0, _CPT, body, 0)
    plsc.subcore_barrier()
    for t in range(_ROWB):
        r = s * _RPT + t * _CHUNK
        pltpu.sync_copy(acc.at[pl.ds(r, _CHUNK)], zbuf)
        pltpu.sync_copy(zbuf, out.at[c, pl.ds(r, _CHUNK)])


@functools.partial(
    pl.kernel,
    out_type=jax.ShapeDtypeStruct((_NC, _NR, _D), jnp.float32),
    mesh=_mesh,
    scratch_types=[
        pltpu.VMEM((_CPT, _CHUNK), jnp.int32),
        pltpu.VMEM((_CPT, _CHUNK), jnp.int32),
        pltpu.VMEM((_CHUNK, _D), jnp.float32),
        pltpu.VMEM_SHARED((_NR, _D), jnp.float32),
        pltpu.SemaphoreType.DMA,
    ],
)
def _sc_edge_sum(g, srcb, dstb, zeros128, out, src_v, dst_v, buf, acc, sem):
    c = lax.axis_index("c")
    s = lax.axis_index("s")
    pltpu.sync_copy(srcb.at[c, s], src_v)
    pltpu.sync_copy(dstb.at[c, s], dst_v)
    pltpu.sync_copy(zeros128, buf)
    for t in range(_ROWB):
        pltpu.sync_copy(buf, acc.at[pl.ds(s * _RPT + t * _CHUNK, _CHUNK)])
    plsc.subcore_barrier()

    def body(j, carry):
        pltpu.async_copy(g.at[src_v.at[j]], buf, sem).wait()
        pltpu.sync_copy(buf, acc.at[dst_v.at[j]], add=True)
        return carry

    lax.fori_loop(0, _CPT, body, 0)
    plsc.subcore_barrier()
    for t in range(_ROWB):
        r = s * _RPT + t * _CHUNK
        pltpu.sync_copy(acc.at[pl.ds(r, _CHUNK)], buf)
        pltpu.sync_copy(buf, out.at[c, pl.ds(r, _CHUNK)])


_BLK = 2000
_GRID = _N // _BLK


def _row_spec(w):
    return pl.BlockSpec((_BLK, w), lambda i: (i, 0))


def _full_spec(shape):
    nd = len(shape)
    return pl.BlockSpec(shape, lambda i: (0,) * nd)


def _prep_body(dp0, dp1, x, dis_o, u0_o):
    deg = dp0[...] + dp1[...]
    rows = lax.broadcasted_iota(jnp.int32, (_BLK, 1), 0)
    first = (pl.program_id(0) == 0) & (rows == 0)
    deg = deg - jnp.where(first, jnp.float32(_NPAD), jnp.float32(0.0))
    dis = jnp.where(deg > 0, 1.0 / jnp.sqrt(jnp.maximum(deg, 1e-12)), 0.0)
    dis_o[...] = dis
    u0_o[...] = x[...] * dis


_prep = pl.pallas_call(
    _prep_body,
    grid=(_GRID,),
    in_specs=[_row_spec(1), _row_spec(1), _row_spec(_D)],
    out_specs=[_row_spec(1), _row_spec(_D)],
    out_shape=[jax.ShapeDtypeStruct((_N, 1), jnp.float32),
               jax.ShapeDtypeStruct((_N, _D), jnp.float32)],
)


def _scale_body(sa, sb, dis_r, tx1_o, u1_o):
    dis = dis_r[...]
    tx1 = -(dis * (sa[...] + sb[...]))
    tx1_o[...] = tx1
    u1_o[...] = dis * tx1


_scale = pl.pallas_call(
    _scale_body,
    grid=(_GRID,),
    in_specs=[_row_spec(_D), _row_spec(_D), _row_spec(1)],
    out_specs=[_row_spec(_D), _row_spec(_D)],
    out_shape=[jax.ShapeDtypeStruct((_N, _D), jnp.float32),
               jax.ShapeDtypeStruct((_N, _D), jnp.float32)],
)


def _mm(a, w):
    return lax.dot_general(a, w, (((1,), (0,)), ((), ())),
                           preferred_element_type=jnp.float32)


def _layer_body(h, tx1, s2a, s2b, dis_r, W, b, hn_o, un_o):
    dis = dis_r[...]
    tx2 = -2.0 * dis * (s2a[...] + s2b[...]) - h[...]
    lin = _mm(h[...], W[0]) + _mm(tx1[...], W[1]) + _mm(tx2, W[2]) + b[...]
    hn = jnp.maximum(lin, 0.0)
    hn_o[...] = hn
    un_o[...] = dis * hn


_layer = pl.pallas_call(
    _layer_body,
    grid=(_GRID,),
    in_specs=[_row_spec(_D), _row_spec(_D), _row_spec(_D), _row_spec(_D),
              _row_spec(1), _full_spec((3, _D, _D)), _full_spec((1, _D))],
    out_specs=[_row_spec(_D), _row_spec(_D)],
    out_shape=[jax.ShapeDtypeStruct((_N, _D), jnp.float32),
               jax.ShapeDtypeStruct((_N, _D), jnp.float32)],
)


def _final_body(h, ty1, s4a, s4b, dis_r, batch, W, b, Wout, bout, out_o,
                sums_acc, cnt_acc):
    i = pl.program_id(0)
    dis = dis_r[...]
    ty2 = -2.0 * dis * (s4a[...] + s4b[...]) - h[...]
    lin = _mm(h[...], W[0]) + _mm(ty1[...], W[1]) + _mm(ty2, W[2]) + b[...]
    h2 = jnp.maximum(lin, 0.0)
    gids = lax.broadcasted_iota(jnp.int32, (1, _NG), 1)
    mask = (batch[...] == gids).astype(jnp.float32)
    psum = lax.dot_general(mask, h2, (((0,), (0,)), ((), ())),
                           preferred_element_type=jnp.float32)
    ones = jnp.ones((_BLK, 1), jnp.float32)
    pcnt = lax.dot_general(mask, ones, (((0,), (0,)), ((), ())),
                           preferred_element_type=jnp.float32)

    @pl.when(i == 0)
    def _():
        sums_acc[...] = jnp.zeros_like(sums_acc)
        cnt_acc[...] = jnp.zeros_like(cnt_acc)

    sums_acc[...] += psum
    cnt_acc[...] += pcnt

    @pl.when(i == _GRID - 1)
    def _():
        cnt = cnt_acc[...]
        mean = jnp.where(cnt > 0, sums_acc[...] / jnp.maximum(cnt, 1.0), 0.0)
        out_o[...] = _mm(mean, Wout[...]) + bout[...]


_final = pl.pallas_call(
    _final_body,
    grid=(_GRID,),
    in_specs=[_row_spec(_D), _row_spec(_D), _row_spec(_D), _row_spec(_D),
              _row_spec(1), _row_spec(1), _full_spec((3, _D, _D)),
              _full_spec((1, _D)), _full_spec((_D, _D)), _full_spec((1, _D))],
    out_specs=pl.BlockSpec((_NG, _D), lambda i: (0, 0)),
    out_shape=jax.ShapeDtypeStruct((_NG, _D), jnp.float32),
    scratch_shapes=[pltpu.VMEM((_NG, _D), jnp.float32),
                    pltpu.VMEM((_NG, 1), jnp.float32)],
)


def kernel(x, edge_index, batch, W1, b1, W2, b2, Wout, bout):
    src = edge_index[0]
    dst = edge_index[1]
    srcp = jnp.concatenate([src, jnp.zeros((_NPAD,), jnp.int32)])
    dstp = jnp.concatenate([dst, jnp.full((_NPAD,), _N, jnp.int32)])
    srcb = srcp.reshape(_NC, _NS, _CPT, _CHUNK)
    dstb = dstp.reshape(_NC, _NS, _CPT, _CHUNK)

    zeros128 = jnp.zeros((_CHUNK, _D), jnp.float32)
    ones128 = jnp.ones((_CHUNK, _D), jnp.float32)

    degp = _sc_degree(srcb, zeros128, ones128)
    dp0 = degp[0, :_N, 0:1]
    dp1 = degp[1, :_N, 0:1]
    dis, u0 = _prep(dp0, dp1, x)

    b1r = b1.reshape(1, _D)
    b2r = b2.reshape(1, _D)
    boutr = bout.reshape(1, _D)
    batch2d = batch.reshape(_N, 1)

    s1 = _sc_edge_sum(u0, srcb, dstb, zeros128)
    tx1, u1 = _scale(s1[0, :_N], s1[1, :_N], dis)
    s2 = _sc_edge_sum(u1, srcb, dstb, zeros128)
    h1, u2 = _layer(x, tx1, s2[0, :_N], s2[1, :_N], dis, W1, b1r)

    s3 = _sc_edge_sum(u2, srcb, dstb, zeros128)
    ty1, u3 = _scale(s3[0, :_N], s3[1, :_N], dis)
    s4 = _sc_edge_sum(u3, srcb, dstb, zeros128)
    out = _final(h1, ty1, s4[0, :_N], s4[1, :_N], dis, batch2d, W2, b2r,
                 Wout, boutr)
    return out
